# baseline (speedup 1.0000x reference)
import jax
import jax.numpy as jnp
from jax.experimental import pallas as pl


def _copy_body(x_ref, o_ref):
    o_ref[...] = x_ref[...]


def kernel(point_cloud):
    b, n, c = point_cloud.shape
    rows = b * n * c // 128
    flat = point_cloud.reshape(rows, 128)
    blk = rows // 8
    out = pl.pallas_call(
        _copy_body,
        grid=(8,),
        in_specs=[pl.BlockSpec((blk, 128), lambda i: (i, 0))],
        out_specs=pl.BlockSpec((blk, 128), lambda i: (i, 0)),
        out_shape=jax.ShapeDtypeStruct((rows, 128), jnp.float32),
    )(flat)
    return out.reshape(b, n, c)


# speedup vs baseline: 43.2186x; 43.2186x over previous
import jax
import jax.numpy as jnp
from jax.experimental import pallas as pl


def _voxel_body(x_ref, o_ref):
    x = x_ref[...]
    mn = jnp.min(x, axis=2, keepdims=True)
    mx = jnp.max(x, axis=2, keepdims=True)
    bw = (mx - mn) / 40.0
    o_ref[...] = jnp.floor((x - mn) / bw)


def kernel(point_cloud):
    b, n, c = point_cloud.shape
    xt = jnp.transpose(point_cloud, (2, 0, 1))
    out = pl.pallas_call(
        _voxel_body,
        out_shape=jax.ShapeDtypeStruct((c, b, n), jnp.float32),
    )(xt)
    return jnp.transpose(out, (1, 2, 0))
